# baseline (speedup 1.0000x reference)
import functools

import jax
import jax.numpy as jnp
from jax import lax
from jax.experimental import pallas as pl
from jax.experimental.pallas import tpu as pltpu
from jax.experimental.pallas import tpu_sc as plsc

_N = 4096
_D = 16
_L = 16
_NC = 2
_NS = 16
_NW = _NC * _NS
_CPAD = 128
_CPW = _CPAD // _NW
_CAP = _N + 2 * _L
_TCAP = 2048
_NBLK = _N // _L


def _rsqrt16(x):
    i = lax.bitcast_convert_type(x, jnp.int32)
    y = lax.bitcast_convert_type(jnp.int32(0x5F3759DF) - (i >> 1), jnp.float32)
    for _ in range(2):
        y = y * (1.5 - 0.5 * x * y * y)
    return y


def _body(emb_hbm, lab_hbm, part_hbm, nu_hbm,
          es_l, lab_l, memb_l, rn_l, est_l, acc_l, nu_l):
    c = lax.axis_index("c")
    s = lax.axis_index("s")
    w = s * _NC + c
    lanes = lax.iota(jnp.int32, _L)
    f0 = jnp.zeros((_L,), jnp.float32)
    i0 = jnp.zeros((_L,), jnp.int32)

    pltpu.sync_copy(lab_hbm, lab_l)
    pltpu.sync_copy(emb_hbm, es_l)

    def scan_blk(tb, curs):
        lv = lab_l[pl.ds(tb * _L, _L)]
        idxv = tb * _L + lanes
        new = []
        for m in range(_CPW):
            hit = lv == (w + m * _NW)
            plsc.store_compressed(memb_l.at[m, pl.ds(curs[m], _L)], idxv,
                                  mask=hit)
            new.append(curs[m] + plsc.all_reduce_population_count(hit)[0])
        return tuple(new)
    cnts = lax.fori_loop(0, _NBLK, scan_blk,
                         tuple(jnp.int32(0) for _ in range(_CPW)))

    for m in range(_CPW):
        memb_l[m, pl.ds(cnts[m], _L)] = i0
        memb_l[m, pl.ds(cnts[m] + _L, _L)] = i0

    def stage_cluster(m, cnt):
        nb = (cnt + _L - 1) >> 4

        def st_blk(b, _):
            rows = memb_l[m, pl.ds(b * _L, _L)]
            ssv = f0
            for k in range(_D):
                colv = plsc.load_gather(
                    es_l, [rows, jnp.full((_L,), k, jnp.int32)])
                est_l[k, pl.ds(b * _L, _L)] = colv
                ssv = ssv + colv * colv
            rn_l[pl.ds(b * _L, _L)] = _rsqrt16(jnp.maximum(ssv, 1e-24))
            return 0
        lax.fori_loop(0, nb, st_blk, 0)

    def pair_block_fast(ii, a, rn_a2, n, jb, acc_v):
        d0 = f0
        d1 = f0
        for k in range(0, _D, 2):
            d0 = d0 + est_l[k, pl.ds(jb * _L, _L)] * a[k]
            d1 = d1 + est_l[k + 1, pl.ds(jb * _L, _L)] * a[k + 1]
        rnv = rn_l[pl.ds(jb * _L, _L)]
        sq = 2.0 - rn_a2 * ((d0 + d1) * rnv)
        sq = jnp.maximum(sq, 1e-30)
        jl = jb * _L + lanes
        valid = (jl > ii) & (jl < n)
        dist = sq * _rsqrt16(sq)
        return acc_v + jnp.where(valid, dist, 0.0)

    def pair_cluster_fast(m, n, acc_v):
        nb = (n + _L - 1) >> 4

        def i_body(ii, acc_v):
            iiv = jnp.full((_L,), ii)
            rn_a = plsc.load_gather(rn_l, [iiv])
            a = [plsc.load_gather(est_l.at[k], [iiv]) for k in range(_D)]
            rn_a2 = rn_a + rn_a
            ib = ii >> 4
            half = (nb - ib + 1) >> 1

            def j2_body(t, acc_v):
                jb = ib + t * 2
                acc_v = pair_block_fast(ii, a, rn_a2, n, jb, acc_v)
                return pair_block_fast(ii, a, rn_a2, n, jb + 1, acc_v)
            return lax.fori_loop(0, half, j2_body, acc_v)

        return lax.fori_loop(0, n, i_body, acc_v)

    def pair_cluster_slow(m, n, acc_v):
        nb = (n + _L - 1) >> 4

        def i_body(ii, acc_v):
            iiv = jnp.full((_L,), ii)
            aidx = plsc.load_gather(memb_l.at[m], [iiv])
            a = [plsc.load_gather(es_l, [aidx, jnp.full((_L,), k, jnp.int32)])
                 for k in range(_D)]
            ssa = f0
            for k in range(_D):
                ssa = ssa + a[k] * a[k]
            rn_a = _rsqrt16(jnp.maximum(ssa, 1e-24))
            rn_a2 = rn_a + rn_a

            def j_body(jb, acc_v):
                rows = memb_l[m, pl.ds(jb * _L, _L)]
                d0 = f0
                for k in range(_D):
                    bk = plsc.load_gather(
                        es_l, [rows, jnp.full((_L,), k, jnp.int32)])
                    d0 = d0 + bk * a[k]
                ssb = f0
                for k in range(_D):
                    bk = plsc.load_gather(
                        es_l, [rows, jnp.full((_L,), k, jnp.int32)])
                    ssb = ssb + bk * bk
                rnv = _rsqrt16(jnp.maximum(ssb, 1e-24))
                sq = 2.0 - rn_a2 * (d0 * rnv)
                sq = jnp.maximum(sq, 1e-30)
                jl = jb * _L + lanes
                valid = (jl > ii) & (jl < n)
                dist = sq * _rsqrt16(sq)
                return acc_v + jnp.where(valid, dist, 0.0)

            return lax.fori_loop(ii >> 4, nb, j_body, acc_v)

        return lax.fori_loop(0, n, i_body, acc_v)

    acc_l[...] = f0
    nun = jnp.int32(0)
    for m in range(_CPW):
        n = cnts[m]

        @pl.when(n <= _TCAP)
        def _fast(m=m, n=n):
            stage_cluster(m, n)
            acc_l[...] = acc_l[...] + pair_cluster_fast(m, n, f0)

        @pl.when(n > _TCAP)
        def _slow(m=m, n=n):
            acc_l[...] = acc_l[...] + pair_cluster_slow(m, n, f0)

        nun = nun + jnp.where(n > 0, 1, 0)

    acc_l[...] = acc_l[...] + acc_l[...]
    nu_l[...] = jnp.where(lanes == 0, jnp.full((_L,), nun), 0
                          ).astype(jnp.float32)
    pltpu.sync_copy(acc_l, part_hbm.at[w])
    pltpu.sync_copy(nu_l, nu_hbm.at[w])


def kernel(embeddings, cluster_labels):
    labels = cluster_labels.astype(jnp.int32)
    mesh = plsc.VectorSubcoreMesh(core_axis_name="c", subcore_axis_name="s",
                                  num_cores=_NC, num_subcores=_NS)
    fn = pl.kernel(
        _body,
        out_type=[
            jax.ShapeDtypeStruct((_NW, _L), jnp.float32),
            jax.ShapeDtypeStruct((_NW, _L), jnp.float32),
        ],
        mesh=mesh,
        compiler_params=pltpu.CompilerParams(needs_layout_passes=False,
                                             use_tc_tiling_on_sc=False),
        scratch_types=[
            pltpu.VMEM((_N, _D), jnp.float32),
            pltpu.VMEM((_N,), jnp.int32),
            pltpu.VMEM((_CPW, _CAP), jnp.int32),
            pltpu.VMEM((_TCAP + 2 * _L,), jnp.float32),
            pltpu.VMEM((_D, _TCAP + 2 * _L), jnp.float32),
            pltpu.VMEM((_L,), jnp.float32),
            pltpu.VMEM((_L,), jnp.float32),
        ],
    )
    part, nu = fn(embeddings, labels)
    return jnp.sum(part) / jnp.sum(nu)


# speedup vs baseline: 1.0557x; 1.0557x over previous
import functools

import jax
import jax.numpy as jnp
from jax import lax
from jax.experimental import pallas as pl
from jax.experimental.pallas import tpu as pltpu
from jax.experimental.pallas import tpu_sc as plsc

_N = 4096
_D = 16
_L = 16
_NC = 2
_NS = 16
_NW = _NC * _NS
_CPAD = 128
_CPW = _CPAD // _NW
_CAP = _N + 2 * _L
_TCAP = 2048
_NBLK = _N // _L
_RS = 2048
_RSB = _RS // _L
_TBLK = 256
_TG = _RS // _TBLK


def _rsqrt16(x):
    i = lax.bitcast_convert_type(x, jnp.int32)
    y = lax.bitcast_convert_type(jnp.int32(0x5F3759DF) - (i >> 1), jnp.float32)
    for _ in range(2):
        y = y * (1.5 - 0.5 * x * y * y)
    return y


def _sc_body(emb_hbm, lab_hbm, part_hbm, nu_hbm,
             es_l, lab_l, memb_l, rn_l, est_l, acc_l, nu_l):
    c = lax.axis_index("c")
    s = lax.axis_index("s")
    w = s * _NC + c
    lanes = lax.iota(jnp.int32, _L)
    f0 = jnp.zeros((_L,), jnp.float32)
    i0 = jnp.zeros((_L,), jnp.int32)

    pltpu.sync_copy(lab_hbm, lab_l)
    pltpu.sync_copy(emb_hbm, es_l)

    def scan_blk(tb, carry):
        curs = carry[:_CPW]
        los = carry[_CPW:]
        lv = lab_l[pl.ds(tb * _L, _L)]
        idxv = tb * _L + lanes
        new = []
        newlo = []
        for m in range(_CPW):
            hit = lv == (w + m * _NW)
            plsc.store_compressed(memb_l.at[m, pl.ds(curs[m], _L)], idxv,
                                  mask=hit)
            pop = plsc.all_reduce_population_count(hit)[0]
            new.append(curs[m] + pop)
            newlo.append(los[m] + jnp.where(tb < _RSB, pop, 0))
        return tuple(new) + tuple(newlo)
    carry = lax.fori_loop(0, _NBLK, scan_blk,
                          tuple(jnp.int32(0) for _ in range(2 * _CPW)))
    cnts = carry[:_CPW]
    lows = carry[_CPW:]

    for m in range(_CPW):
        memb_l[m, pl.ds(cnts[m], _L)] = i0
        memb_l[m, pl.ds(cnts[m] + _L, _L)] = i0

    def stage_cluster(m, cnt):
        nb = (cnt + _L - 1) >> 4

        def st_blk(b, _):
            rows = memb_l[m, pl.ds(b * _L, _L)]
            ssv = f0
            for k in range(_D):
                colv = plsc.load_gather(
                    es_l, [rows, jnp.full((_L,), k, jnp.int32)])
                est_l[k, pl.ds(b * _L, _L)] = colv
                ssv = ssv + colv * colv
            rn_l[pl.ds(b * _L, _L)] = _rsqrt16(jnp.maximum(ssv, 1e-24))
            return 0
        lax.fori_loop(0, nb, st_blk, 0)

    def pair_block_fast(ii, a, rn_a2, n, jb, acc_v):
        d0 = f0
        d1 = f0
        for k in range(0, _D, 2):
            d0 = d0 + est_l[k, pl.ds(jb * _L, _L)] * a[k]
            d1 = d1 + est_l[k + 1, pl.ds(jb * _L, _L)] * a[k + 1]
        rnv = rn_l[pl.ds(jb * _L, _L)]
        sq = 2.0 - rn_a2 * ((d0 + d1) * rnv)
        sq = jnp.maximum(sq, 1e-30)
        jl = jb * _L + lanes
        valid = (jl > ii) & (jl < n)
        dist = sq * _rsqrt16(sq)
        return acc_v + jnp.where(valid, dist, 0.0)

    def pair_cluster_fast(m, n, n_lo, acc_v):
        nb = (n + _L - 1) >> 4

        def i_body(ii, acc_v):
            iiv = jnp.full((_L,), ii)
            rn_a = plsc.load_gather(rn_l, [iiv])
            a = [plsc.load_gather(est_l.at[k], [iiv]) for k in range(_D)]
            rn_a2 = rn_a + rn_a
            ib = ii >> 4
            half = (nb - ib + 1) >> 1

            def j2_body(t, acc_v):
                jb = ib + t * 2
                acc_v = pair_block_fast(ii, a, rn_a2, n, jb, acc_v)
                return pair_block_fast(ii, a, rn_a2, n, jb + 1, acc_v)
            return lax.fori_loop(0, half, j2_body, acc_v)

        return lax.fori_loop(n_lo, n, i_body, acc_v)

    def pair_cluster_slow(m, n, n_lo, acc_v):
        nb = (n + _L - 1) >> 4

        def i_body(ii, acc_v):
            iiv = jnp.full((_L,), ii)
            aidx = plsc.load_gather(memb_l.at[m], [iiv])
            a = [plsc.load_gather(es_l, [aidx, jnp.full((_L,), k, jnp.int32)])
                 for k in range(_D)]
            ssa = f0
            for k in range(_D):
                ssa = ssa + a[k] * a[k]
            rn_a = _rsqrt16(jnp.maximum(ssa, 1e-24))
            rn_a2 = rn_a + rn_a

            def j_body(jb, acc_v):
                rows = memb_l[m, pl.ds(jb * _L, _L)]
                d0 = f0
                for k in range(_D):
                    bk = plsc.load_gather(
                        es_l, [rows, jnp.full((_L,), k, jnp.int32)])
                    d0 = d0 + bk * a[k]
                ssb = f0
                for k in range(_D):
                    bk = plsc.load_gather(
                        es_l, [rows, jnp.full((_L,), k, jnp.int32)])
                    ssb = ssb + bk * bk
                rnv = _rsqrt16(jnp.maximum(ssb, 1e-24))
                sq = 2.0 - rn_a2 * (d0 * rnv)
                sq = jnp.maximum(sq, 1e-30)
                jl = jb * _L + lanes
                valid = (jl > ii) & (jl < n)
                dist = sq * _rsqrt16(sq)
                return acc_v + jnp.where(valid, dist, 0.0)

            return lax.fori_loop(ii >> 4, nb, j_body, acc_v)

        return lax.fori_loop(n_lo, n, i_body, acc_v)

    acc_l[...] = f0
    nun = jnp.int32(0)
    for m in range(_CPW):
        n = cnts[m]
        n_lo = lows[m]

        @pl.when(n <= _TCAP)
        def _fast(m=m, n=n, n_lo=n_lo):
            stage_cluster(m, n)
            acc_l[...] = acc_l[...] + pair_cluster_fast(m, n, n_lo, f0)

        @pl.when(n > _TCAP)
        def _slow(m=m, n=n, n_lo=n_lo):
            acc_l[...] = acc_l[...] + pair_cluster_slow(m, n, n_lo, f0)

        nun = nun + jnp.where(n > 0, 1, 0)

    acc_l[...] = acc_l[...] + acc_l[...]
    nu_l[...] = jnp.where(lanes == 0, jnp.full((_L,), nun), 0
                          ).astype(jnp.float32)
    pltpu.sync_copy(acc_l, part_hbm.at[w])
    pltpu.sync_copy(nu_l, nu_hbm.at[w])


def _tc_body(e_ref, labf_ref, labc_ref, out_ref, en_ref, acc_ref):
    i = pl.program_id(0)

    @pl.when(i == 0)
    def _init():
        e = e_ref[...]
        ss = jnp.sum(e * e, axis=1, keepdims=True)
        inv = jax.lax.rsqrt(jnp.maximum(ss, 1e-24))
        en_ref[...] = e * inv
        acc_ref[0, 0] = 0.0

    en = en_ref[...]
    rows = en_ref[pl.ds(i * _TBLK, _TBLK), :]
    g = jax.lax.dot_general(rows, en, (((1,), (1,)), ((), ())),
                            preferred_element_type=jnp.float32)
    sqr = jnp.sum(rows * rows, axis=1, keepdims=True)
    sqa = jnp.sum(en * en, axis=1)[None, :]
    sq = jnp.maximum(sqr + sqa - 2.0 * g, 0.0)
    dist = jnp.sqrt(sq)
    wcol = 1.0 + (jax.lax.broadcasted_iota(jnp.int32, (1, _N), 1)
                  >= _RS).astype(jnp.float32)
    mask = labc_ref[...] == labf_ref[...]
    acc_ref[0, 0] += jnp.sum(jnp.where(mask, dist * wcol, 0.0))

    @pl.when(i == _TG - 1)
    def _fin():
        out_ref[...] = jnp.full((1, 1), acc_ref[0, 0], dtype=jnp.float32)


def kernel(embeddings, cluster_labels):
    labels = cluster_labels.astype(jnp.int32)
    mesh = plsc.VectorSubcoreMesh(core_axis_name="c", subcore_axis_name="s",
                                  num_cores=_NC, num_subcores=_NS)
    sc_fn = pl.kernel(
        _sc_body,
        out_type=[
            jax.ShapeDtypeStruct((_NW, _L), jnp.float32),
            jax.ShapeDtypeStruct((_NW, _L), jnp.float32),
        ],
        mesh=mesh,
        compiler_params=pltpu.CompilerParams(needs_layout_passes=False,
                                             use_tc_tiling_on_sc=False),
        scratch_types=[
            pltpu.VMEM((_N, _D), jnp.float32),
            pltpu.VMEM((_N,), jnp.int32),
            pltpu.VMEM((_CPW, _CAP), jnp.int32),
            pltpu.VMEM((_TCAP + 2 * _L,), jnp.float32),
            pltpu.VMEM((_D, _TCAP + 2 * _L), jnp.float32),
            pltpu.VMEM((_L,), jnp.float32),
            pltpu.VMEM((_L,), jnp.float32),
        ],
    )
    part, nu = sc_fn(embeddings, labels)

    tc_sum = pl.pallas_call(
        _tc_body,
        grid=(_TG,),
        in_specs=[
            pl.BlockSpec((_N, _D), lambda i: (0, 0)),
            pl.BlockSpec((1, _N), lambda i: (0, 0)),
            pl.BlockSpec((_TBLK, 1), lambda i: (i, 0)),
        ],
        out_specs=pl.BlockSpec((1, 1), lambda i: (0, 0)),
        out_shape=jax.ShapeDtypeStruct((1, 1), jnp.float32),
        scratch_shapes=[
            pltpu.VMEM((_N, _D), jnp.float32),
            pltpu.SMEM((1, 1), jnp.float32),
        ],
    )(embeddings, labels.reshape(1, _N), labels.reshape(_N, 1))

    return (tc_sum[0, 0] + jnp.sum(part)) / jnp.sum(nu)
